# baseline (speedup 1.0000x reference)
import functools

import numpy as np

import jax
import jax.numpy as jnp
from jax import lax
from jax.experimental import pallas as pl
from jax.experimental.pallas import tpu as pltpu
from jax.experimental.pallas import tpu_sc as plsc

T, C, H, DH = 2048, 768, 12, 64
D = 64
KVW = 2 * DH
NC, NS, L = 2, 16, 16
NW = NC * NS
TPW = T // NW

_mesh = plsc.VectorSubcoreMesh(
    core_axis_name="c", subcore_axis_name="s", num_cores=NC, num_subcores=NS)

_EV = np.arange(0, 2 * L, 2)
_PERM64 = np.concatenate([_EV, _EV + 1, 2 * L + _EV, 2 * L + _EV + 1])
_PERMC = np.concatenate([h * DH + _PERM64 for h in range(H)])


_DN = (((1,), (1,)), ((), ()))


def _qkv_body(x_ref, w_ref, q_ref, kv_ref):
    a = x_ref[...]
    for h in range(H):
        q_ref[h] = lax.dot_general(
            a, w_ref[pl.ds(h * DH, DH), :], _DN,
            preferred_element_type=jnp.float32)
        kv_ref[h, :, 0:DH] = lax.dot_general(
            a, w_ref[pl.ds(C + h * DH, DH), :], _DN,
            preferred_element_type=jnp.float32).astype(jnp.bfloat16)
        kv_ref[h, :, DH:KVW] = lax.dot_general(
            a, w_ref[pl.ds(2 * C + h * DH, DH), :], _DN,
            preferred_element_type=jnp.float32).astype(jnp.bfloat16)


def _qkv_proj(x, w, bm=256):
    return pl.pallas_call(
        _qkv_body,
        grid=(T // bm,),
        in_specs=[pl.BlockSpec((bm, C), lambda i: (i, 0)),
                  pl.BlockSpec((3 * C, C), lambda i: (0, 0))],
        out_specs=[pl.BlockSpec((H, bm, DH), lambda i: (0, i, 0)),
                   pl.BlockSpec((H, bm, KVW), lambda i: (0, i, 0))],
        out_shape=[jax.ShapeDtypeStruct((H, T, DH), jnp.float32),
                   jax.ShapeDtypeStruct((H, T, KVW), jnp.bfloat16)],
    )(x, w)


def _oproj_body(a_ref, w_ref, o_ref):
    acc = jnp.zeros_like(o_ref)
    for h in range(H):
        acc = acc + lax.dot_general(
            a_ref[h], w_ref[:, pl.ds(h * DH, DH)], _DN,
            preferred_element_type=jnp.float32)
    o_ref[...] = acc


def _out_proj(attn, w, bm=256):
    return pl.pallas_call(
        _oproj_body,
        grid=(T // bm,),
        in_specs=[pl.BlockSpec((H, bm, DH), lambda i: (0, i, 0)),
                  pl.BlockSpec((C, C), lambda i: (0, 0))],
        out_specs=pl.BlockSpec((bm, C), lambda i: (i, 0)),
        out_shape=jax.ShapeDtypeStruct((T, C), jnp.float32),
    )(attn, w)


def _attn_body(q_hbm, kv_hbm, ng_hbm, out_hbm, qs, ns, kvga, kvgb, kvs,
               tbuf, outs, sema, semb):
    wid = lax.axis_index("s") * NC + lax.axis_index("c")
    t0 = wid * TPW
    sid = lax.axis_index("s")

    def stage_body(h, carry):
        pltpu.sync_copy(kv_hbm.at[h, pl.ds(sid * (T // NS), T // NS)],
                        kvs.at[h, pl.ds(sid * (T // NS), T // NS)])
        return carry

    lax.fori_loop(0, H, stage_body, 0)
    plsc.subcore_barrier()
    iota = lax.broadcasted_iota(jnp.int32, (L,), 0)
    NEG = jnp.float32(-1e30)

    def split_pairs(chunk_bf16):
        kb = plsc.bitcast(chunk_bf16, jnp.int32)
        ev = plsc.bitcast(kb << 16, jnp.float32)
        od = plsc.bitcast(kb, jnp.float32)
        return ev, od

    def compute_token(ti, kvg):
        t = t0 + ti
        roff = 0
        qv = [qs[ti, pl.ds(c * L, L)] for c in range(4)]
        acc = []
        for wc in range(4):
            for lane in range(L):
                j = wc * L + lane
                k0e, k0o = split_pairs(kvg[roff + j, pl.ds(0, 2 * L)])
                k1e, k1o = split_pairs(kvg[roff + j, pl.ds(2 * L, 2 * L)])
                d0 = ((k0e * qv[0] + k0o * qv[1])
                      + (k1e * qv[2] + k1o * qv[3]))
                tbuf[lane, pl.ds(0, L)] = d0
            sv = plsc.load_gather(tbuf, [iota, jnp.full((L,), 0, jnp.int32)])
            for col in range(1, L):
                sv = sv + plsc.load_gather(
                    tbuf, [iota, jnp.full((L,), jnp.int32(col), jnp.int32)])
            acc.append(sv)

        masks = [ns[ti, pl.ds(c * L, L)] <= t for c in range(4)]
        mvecs = [jnp.where(masks[c], acc[c] * jnp.float32(0.125), NEG)
                 for c in range(4)]
        mx = jnp.max(jnp.maximum(jnp.maximum(mvecs[0], mvecs[1]),
                                 jnp.maximum(mvecs[2], mvecs[3])))
        mx = jnp.where(mx > jnp.float32(-5e29), mx, jnp.float32(0.0))
        evecs = [jnp.where(masks[c], jnp.exp(mvecs[c] - mx),
                           jnp.float32(0.0)) for c in range(4)]
        ssum = jnp.sum(evecs[0] + evecs[1] + evecs[2] + evecs[3])
        winv = jnp.ones((L,), jnp.float32) / jnp.maximum(
            jnp.full((L,), ssum, jnp.float32), jnp.float32(1e-9))
        wvecs = [evecs[c] * winv for c in range(4)]

        oacc = [jnp.zeros((L,), jnp.float32) for _ in range(4)]
        for wc in range(4):
            for lane in range(L):
                j = wc * L + lane
                wb = jnp.full((L,), wvecs[wc][lane], jnp.float32)
                v0e, v0o = split_pairs(kvg[roff + j, pl.ds(DH, 2 * L)])
                v1e, v1o = split_pairs(kvg[roff + j, pl.ds(DH + 2 * L, 2 * L)])
                oacc[0] = oacc[0] + wb * v0e
                oacc[1] = oacc[1] + wb * v0o
                oacc[2] = oacc[2] + wb * v1e
                oacc[3] = oacc[3] + wb * v1o
        for c in range(4):
            outs[ti, pl.ds(c * L, L)] = oacc[c]

    def head_body(h, carry):
        pltpu.sync_copy(q_hbm.at[h, pl.ds(t0, TPW)], qs)
        pltpu.sync_copy(ng_hbm.at[h, pl.ds(t0, TPW)], ns)

        def gather(ti, buf, sem):
            pltpu.make_async_copy(kvs.at[h].at[ns.at[ti]], buf,
                                  sem).start()

        gather(0, kvga, sema)

        def tok_body(ti2, carry2):
            ta = 2 * ti2
            gather(ta + 1, kvgb, semb)
            pltpu.make_async_copy(kvs.at[h].at[ns.at[ta]], kvga,
                                  sema).wait()
            compute_token(ta, kvga)
            gather(jnp.minimum(ta + 2, TPW - 1), kvga, sema)
            pltpu.make_async_copy(kvs.at[h].at[ns.at[ta + 1]], kvgb,
                                  semb).wait()
            compute_token(ta + 1, kvgb)
            return carry2

        lax.fori_loop(0, TPW // 2, tok_body, 0)
        pltpu.make_async_copy(kvs.at[h].at[ns.at[TPW - 1]], kvga,
                              sema).wait()
        pltpu.sync_copy(outs, out_hbm.at[h, pl.ds(t0, TPW)])
        return carry

    lax.fori_loop(0, H, head_body, 0)


_sc_attn = functools.partial(
    pl.kernel,
    out_type=jax.ShapeDtypeStruct((H, T, DH), jnp.float32),
    mesh=_mesh,
    compiler_params=pltpu.CompilerParams(
        needs_layout_passes=False, use_tc_tiling_on_sc=False),
    scratch_types=[
        pltpu.VMEM((TPW, DH), jnp.float32),
        pltpu.VMEM((TPW, D), jnp.int32),
        pltpu.VMEM((D, KVW), jnp.bfloat16),
        pltpu.VMEM((D, KVW), jnp.bfloat16),
        pltpu.VMEM_SHARED((H, T, KVW), jnp.bfloat16),
        pltpu.VMEM((L, L + 1), jnp.float32),
        pltpu.VMEM((TPW, DH), jnp.float32),
        pltpu.SemaphoreType.DMA,
        pltpu.SemaphoreType.DMA,
    ],
)(_attn_body)


def kernel(x, neigh_idx, Wqkv, Wout):
    x2 = x[0]
    w2 = jnp.concatenate([Wqkv[:C][_PERMC], Wqkv[C:]], axis=0)
    wo2 = Wout[:, _PERMC]
    q, kv = _qkv_proj(x2, w2)
    attn = _sc_attn(q, kv, neigh_idx.astype(jnp.int32))
    y = _out_proj(attn, wo2)
    return y[None]


# speedup vs baseline: 1.2558x; 1.2558x over previous
import functools

import numpy as np

import jax
import jax.numpy as jnp
from jax import lax
from jax.experimental import pallas as pl
from jax.experimental.pallas import tpu as pltpu
from jax.experimental.pallas import tpu_sc as plsc

T, C, H, DH = 2048, 768, 12, 64
D = 64
KVW = 2 * DH
NC, NS, L = 2, 16, 16
NW = NC * NS
TPW = T // NW

_mesh = plsc.VectorSubcoreMesh(
    core_axis_name="c", subcore_axis_name="s", num_cores=NC, num_subcores=NS)

_EV = np.arange(0, 2 * L, 2)
_PERM64 = np.concatenate([_EV, _EV + 1, 2 * L + _EV, 2 * L + _EV + 1])
_PERMC = np.concatenate([h * DH + _PERM64 for h in range(H)])


_DN = (((1,), (1,)), ((), ()))


def _qkv_body(x_ref, w_ref, q_ref, kv_ref):
    a = x_ref[...]
    for h in range(H):
        q_ref[h] = lax.dot_general(
            a, w_ref[pl.ds(h * DH, DH), :], _DN,
            preferred_element_type=jnp.float32)
        kv_ref[h, :, 0:DH] = lax.dot_general(
            a, w_ref[pl.ds(C + h * DH, DH), :], _DN,
            preferred_element_type=jnp.float32).astype(jnp.bfloat16)
        kv_ref[h, :, DH:KVW] = lax.dot_general(
            a, w_ref[pl.ds(2 * C + h * DH, DH), :], _DN,
            preferred_element_type=jnp.float32).astype(jnp.bfloat16)


def _qkv_proj(x, w, bm=256):
    return pl.pallas_call(
        _qkv_body,
        grid=(T // bm,),
        in_specs=[pl.BlockSpec((bm, C), lambda i: (i, 0)),
                  pl.BlockSpec((3 * C, C), lambda i: (0, 0))],
        out_specs=[pl.BlockSpec((H, bm, DH), lambda i: (0, i, 0)),
                   pl.BlockSpec((H, bm, KVW), lambda i: (0, i, 0))],
        out_shape=[jax.ShapeDtypeStruct((H, T, DH), jnp.float32),
                   jax.ShapeDtypeStruct((H, T, KVW), jnp.bfloat16)],
    )(x, w)


def _oproj_body(a_ref, w_ref, o_ref):
    acc = jnp.zeros_like(o_ref)
    for h in range(H):
        acc = acc + lax.dot_general(
            a_ref[h], w_ref[:, pl.ds(h * DH, DH)], _DN,
            preferred_element_type=jnp.float32)
    o_ref[...] = acc


def _out_proj(attn, w, bm=256):
    return pl.pallas_call(
        _oproj_body,
        grid=(T // bm,),
        in_specs=[pl.BlockSpec((H, bm, DH), lambda i: (0, i, 0)),
                  pl.BlockSpec((C, C), lambda i: (0, 0))],
        out_specs=pl.BlockSpec((bm, C), lambda i: (i, 0)),
        out_shape=jax.ShapeDtypeStruct((T, C), jnp.float32),
    )(attn, w)


def _attn_body(q_hbm, kv_hbm, ng_hbm, out_hbm, qs, ns, kvga, kvgb, kvs,
               tbuf, outs, sema, semb):
    wid = lax.axis_index("s") * NC + lax.axis_index("c")
    t0 = wid * TPW
    sid = lax.axis_index("s")

    def stage_body(h, carry):
        pltpu.sync_copy(kv_hbm.at[h, pl.ds(sid * (T // NS), T // NS)],
                        kvs.at[h, pl.ds(sid * (T // NS), T // NS)])
        return carry

    lax.fori_loop(0, H, stage_body, 0)
    plsc.subcore_barrier()
    iota = lax.broadcasted_iota(jnp.int32, (L,), 0)
    NEG = jnp.float32(-1e30)

    def split_pairs(chunk_bf16):
        kb = plsc.bitcast(chunk_bf16, jnp.int32)
        ev = plsc.bitcast(kb << 16, jnp.float32)
        od = plsc.bitcast(kb, jnp.float32)
        return ev, od

    def compute_token(ti, kvg):
        t = t0 + ti
        roff = 0
        qv = [qs[ti, pl.ds(c * L, L)] for c in range(4)]
        acc = []
        for wc in range(4):
            sv = jnp.zeros((L,), jnp.float32)
            for lane in range(L):
                j = wc * L + lane
                k0e, k0o = split_pairs(kvg[roff + j, pl.ds(0, 2 * L)])
                k1e, k1o = split_pairs(kvg[roff + j, pl.ds(2 * L, 2 * L)])
                d0 = ((k0e * qv[0] + k0o * qv[1])
                      + (k1e * qv[2] + k1o * qv[3]))
                s = jnp.sum(d0)
                sv = jnp.where(iota == jnp.int32(lane),
                               jnp.full((L,), s, jnp.float32), sv)
            acc.append(sv)

        masks = [ns[ti, pl.ds(c * L, L)] <= t for c in range(4)]
        mvecs = [jnp.where(masks[c], acc[c] * jnp.float32(0.125), NEG)
                 for c in range(4)]
        mx = jnp.max(jnp.maximum(jnp.maximum(mvecs[0], mvecs[1]),
                                 jnp.maximum(mvecs[2], mvecs[3])))
        mx = jnp.where(mx > jnp.float32(-5e29), mx, jnp.float32(0.0))
        evecs = [jnp.where(masks[c], jnp.exp(mvecs[c] - mx),
                           jnp.float32(0.0)) for c in range(4)]
        ssum = jnp.sum(evecs[0] + evecs[1] + evecs[2] + evecs[3])
        winv = jnp.ones((L,), jnp.float32) / jnp.maximum(
            jnp.full((L,), ssum, jnp.float32), jnp.float32(1e-9))
        wvecs = [evecs[c] * winv for c in range(4)]

        oacc = [jnp.zeros((L,), jnp.float32) for _ in range(4)]
        for wc in range(4):
            for lane in range(L):
                j = wc * L + lane
                wb = jnp.full((L,), wvecs[wc][lane], jnp.float32)
                v0e, v0o = split_pairs(kvg[roff + j, pl.ds(DH, 2 * L)])
                v1e, v1o = split_pairs(kvg[roff + j, pl.ds(DH + 2 * L, 2 * L)])
                oacc[0] = oacc[0] + wb * v0e
                oacc[1] = oacc[1] + wb * v0o
                oacc[2] = oacc[2] + wb * v1e
                oacc[3] = oacc[3] + wb * v1o
        for c in range(4):
            outs[ti, pl.ds(c * L, L)] = oacc[c]

    def head_body(h, carry):
        pltpu.sync_copy(q_hbm.at[h, pl.ds(t0, TPW)], qs)
        pltpu.sync_copy(ng_hbm.at[h, pl.ds(t0, TPW)], ns)

        def gather(ti, buf, sem):
            pltpu.make_async_copy(kvs.at[h].at[ns.at[ti]], buf,
                                  sem).start()

        gather(0, kvga, sema)

        def tok_body(ti2, carry2):
            ta = 2 * ti2
            gather(ta + 1, kvgb, semb)
            pltpu.make_async_copy(kvs.at[h].at[ns.at[ta]], kvga,
                                  sema).wait()
            compute_token(ta, kvga)
            gather(jnp.minimum(ta + 2, TPW - 1), kvga, sema)
            pltpu.make_async_copy(kvs.at[h].at[ns.at[ta + 1]], kvgb,
                                  semb).wait()
            compute_token(ta + 1, kvgb)
            return carry2

        lax.fori_loop(0, TPW // 2, tok_body, 0)
        pltpu.make_async_copy(kvs.at[h].at[ns.at[TPW - 1]], kvga,
                              sema).wait()
        pltpu.sync_copy(outs, out_hbm.at[h, pl.ds(t0, TPW)])
        return carry

    lax.fori_loop(0, H, head_body, 0)


_sc_attn = functools.partial(
    pl.kernel,
    out_type=jax.ShapeDtypeStruct((H, T, DH), jnp.float32),
    mesh=_mesh,
    compiler_params=pltpu.CompilerParams(
        needs_layout_passes=False, use_tc_tiling_on_sc=False),
    scratch_types=[
        pltpu.VMEM((TPW, DH), jnp.float32),
        pltpu.VMEM((TPW, D), jnp.int32),
        pltpu.VMEM((D, KVW), jnp.bfloat16),
        pltpu.VMEM((D, KVW), jnp.bfloat16),
        pltpu.VMEM_SHARED((H, T, KVW), jnp.bfloat16),
        pltpu.VMEM((L, L + 1), jnp.float32),
        pltpu.VMEM((TPW, DH), jnp.float32),
        pltpu.SemaphoreType.DMA,
        pltpu.SemaphoreType.DMA,
    ],
)(_attn_body)


def kernel(x, neigh_idx, Wqkv, Wout):
    x2 = x[0]
    w2 = jnp.concatenate([Wqkv[:C][_PERMC], Wqkv[C:]], axis=0)
    wo2 = Wout[:, _PERMC]
    q, kv = _qkv_proj(x2, w2)
    attn = _sc_attn(q, kv, neigh_idx.astype(jnp.int32))
    y = _out_proj(attn, wo2)
    return y[None]
